# baseline (device time: 16825 ns/iter reference)
import jax
import jax.numpy as jnp
from jax import lax
from jax.experimental import pallas as pl
from jax.experimental.pallas import tpu as pltpu

M = 512
D = 512
HALF = M // 2
NCHUNK = 8
CROWS = HALF // NCHUNK


def kernel(partial, resid, gamma):
    def body(p_ref, r_ref, g_ref, o_ref, comm_ref, t_ref, ob_ref,
             send_a, recv_a, send_b, recv_b):
        my_x = lax.axis_index("x")
        my_y = lax.axis_index("y")
        my_z = lax.axis_index("z")
        y_nbr = (my_x, 1 - my_y, my_z)
        x_nbr = (1 - my_x, my_y, my_z)

        barrier = pltpu.get_barrier_semaphore()
        for nbr in (y_nbr, x_nbr):
            pl.semaphore_signal(
                barrier, inc=1, device_id=nbr,
                device_id_type=pl.DeviceIdType.MESH,
            )
        pl.semaphore_wait(barrier, 2)

        base = my_x * HALF

        rdma_a = []
        for c in range(NCHUNK):
            rows = pl.ds(base + c * CROWS, CROWS)
            a = pltpu.make_async_remote_copy(
                src_ref=p_ref.at[0, rows],
                dst_ref=comm_ref.at[c],
                send_sem=send_a.at[c],
                recv_sem=recv_a.at[c],
                device_id=y_nbr,
                device_id_type=pl.DeviceIdType.MESH,
            )
            a.start()
            rdma_a.append(a)

        t_ref[...] = p_ref[0, pl.ds(base, HALF)] + r_ref[pl.ds(base, HALF)]

        rdma_b = []
        for c in range(NCHUNK):
            rows = pl.ds(base + c * CROWS, CROWS)
            rdma_a[c].wait_recv()
            y = t_ref[pl.ds(c * CROWS, CROWS)] + comm_ref[c]
            ms = jnp.mean(y * y, axis=-1, keepdims=True)
            ob_ref[c] = y * lax.rsqrt(ms + 1e-6) * g_ref[...]
            b = pltpu.make_async_remote_copy(
                src_ref=ob_ref.at[c],
                dst_ref=o_ref.at[rows],
                send_sem=send_b.at[c],
                recv_sem=recv_b.at[c],
                device_id=x_nbr,
                device_id_type=pl.DeviceIdType.MESH,
            )
            b.start()
            rdma_b.append(b)
            o_ref[rows] = ob_ref[c]

        for c in range(NCHUNK):
            rdma_a[c].wait_send()
        for c in range(NCHUNK):
            rdma_b[c].wait()

    return pl.pallas_call(
        body,
        out_shape=jax.ShapeDtypeStruct((M, D), jnp.float32),
        in_specs=[
            pl.BlockSpec(memory_space=pltpu.VMEM),
            pl.BlockSpec(memory_space=pltpu.VMEM),
            pl.BlockSpec(memory_space=pltpu.VMEM),
        ],
        out_specs=pl.BlockSpec(memory_space=pltpu.VMEM),
        scratch_shapes=[
            pltpu.VMEM((NCHUNK, CROWS, D), jnp.float32),
            pltpu.VMEM((HALF, D), jnp.float32),
            pltpu.VMEM((NCHUNK, CROWS, D), jnp.float32),
            pltpu.SemaphoreType.DMA((NCHUNK,)),
            pltpu.SemaphoreType.DMA((NCHUNK,)),
            pltpu.SemaphoreType.DMA((NCHUNK,)),
            pltpu.SemaphoreType.DMA((NCHUNK,)),
        ],
        compiler_params=pltpu.CompilerParams(collective_id=0),
    )(partial, resid, gamma)


# device time: 16789 ns/iter; 1.0021x vs baseline; 1.0021x over previous
import jax
import jax.numpy as jnp
from jax import lax
from jax.experimental import pallas as pl
from jax.experimental.pallas import tpu as pltpu

M = 512
D = 512
HALF = M // 2
NCHUNK = 8
CROWS = HALF // NCHUNK


def kernel(partial, resid, gamma):
    def body(p_ref, r_ref, g_ref, o_ref, comm_ref, t_ref, ob_ref,
             send_a, recv_a, send_b, recv_b):
        my_x = lax.axis_index("x")
        my_y = lax.axis_index("y")
        my_z = lax.axis_index("z")
        y_nbr = (my_x, 1 - my_y, my_z)
        x_nbr = (1 - my_x, my_y, my_z)

        barrier = pltpu.get_barrier_semaphore()
        for nbr in (y_nbr, x_nbr):
            pl.semaphore_signal(
                barrier, inc=1, device_id=nbr,
                device_id_type=pl.DeviceIdType.MESH,
            )
        pl.semaphore_wait(barrier, 2)

        base = my_x * HALF

        rdma_a = []
        for c in range(NCHUNK):
            rows = pl.ds(base + c * CROWS, CROWS)
            a = pltpu.make_async_remote_copy(
                src_ref=p_ref.at[0, rows],
                dst_ref=comm_ref.at[c],
                send_sem=send_a.at[c],
                recv_sem=recv_a.at[c],
                device_id=y_nbr,
                device_id_type=pl.DeviceIdType.MESH,
            )
            a.start()
            rdma_a.append(a)

        t_ref[...] = p_ref[0, pl.ds(base, HALF)] + r_ref[pl.ds(base, HALF)]

        rdma_b = []
        for c in range(NCHUNK):
            rows = pl.ds(base + c * CROWS, CROWS)
            rdma_a[c].wait_recv()
            y = t_ref[pl.ds(c * CROWS, CROWS)] + comm_ref[c]
            ms = jnp.mean(y * y, axis=-1, keepdims=True)
            ob_ref[c] = y * lax.rsqrt(ms + 1e-6) * g_ref[...]
            b = pltpu.make_async_remote_copy(
                src_ref=ob_ref.at[c],
                dst_ref=o_ref.at[rows],
                send_sem=send_b.at[c],
                recv_sem=recv_b.at[c],
                device_id=x_nbr,
                device_id_type=pl.DeviceIdType.MESH,
            )
            b.start()
            rdma_b.append(b)

        o_ref[pl.ds(base, HALF)] = ob_ref[...].reshape(HALF, D)

        for c in range(NCHUNK):
            rdma_a[c].wait_send()
        for c in range(NCHUNK):
            rdma_b[c].wait()

    return pl.pallas_call(
        body,
        out_shape=jax.ShapeDtypeStruct((M, D), jnp.float32),
        in_specs=[
            pl.BlockSpec(memory_space=pltpu.VMEM),
            pl.BlockSpec(memory_space=pltpu.VMEM),
            pl.BlockSpec(memory_space=pltpu.VMEM),
        ],
        out_specs=pl.BlockSpec(memory_space=pltpu.VMEM),
        scratch_shapes=[
            pltpu.VMEM((NCHUNK, CROWS, D), jnp.float32),
            pltpu.VMEM((HALF, D), jnp.float32),
            pltpu.VMEM((NCHUNK, CROWS, D), jnp.float32),
            pltpu.SemaphoreType.DMA((NCHUNK,)),
            pltpu.SemaphoreType.DMA((NCHUNK,)),
            pltpu.SemaphoreType.DMA((NCHUNK,)),
            pltpu.SemaphoreType.DMA((NCHUNK,)),
        ],
        compiler_params=pltpu.CompilerParams(collective_id=0),
    )(partial, resid, gamma)


# device time: 15024 ns/iter; 1.1199x vs baseline; 1.1175x over previous
import os

import jax
import jax.numpy as jnp
from jax import lax
from jax.experimental import pallas as pl
from jax.experimental.pallas import tpu as pltpu

M = 512
D = 512
HALF = M // 2
NCHUNK = 8
CROWS = HALF // NCHUNK
PROBE = os.environ.get("PROBE", "full")


def kernel(partial, resid, gamma):
    def body(p_ref, r_ref, g_ref, o_ref, comm_ref, t_ref, ob_ref,
             send_a, recv_a, send_b, recv_b):
        my_x = lax.axis_index("x")
        my_y = lax.axis_index("y")
        my_z = lax.axis_index("z")
        y_nbr = (my_x, 1 - my_y, my_z)
        x_nbr = (1 - my_x, my_y, my_z)

        barrier = pltpu.get_barrier_semaphore()
        for nbr in (y_nbr, x_nbr):
            pl.semaphore_signal(
                barrier, inc=1, device_id=nbr,
                device_id_type=pl.DeviceIdType.MESH,
            )
        pl.semaphore_wait(barrier, 2)

        base = my_x * HALF

        if PROBE == "barrier":
            y = p_ref[0, pl.ds(base, HALF)] + r_ref[pl.ds(base, HALF)]
            ms = jnp.mean(y * y, axis=-1, keepdims=True)
            o_ref[pl.ds(base, HALF)] = y * lax.rsqrt(ms + 1e-6) * g_ref[...]
            o_ref[pl.ds((1 - my_x) * HALF, HALF)] = y
            return

        rdma_a = []
        for c in range(NCHUNK):
            rows = pl.ds(base + c * CROWS, CROWS)
            a = pltpu.make_async_remote_copy(
                src_ref=p_ref.at[0, rows],
                dst_ref=comm_ref.at[c],
                send_sem=send_a.at[c],
                recv_sem=recv_a.at[c],
                device_id=y_nbr,
                device_id_type=pl.DeviceIdType.MESH,
            )
            a.start()
            rdma_a.append(a)

        t_ref[...] = p_ref[0, pl.ds(base, HALF)] + r_ref[pl.ds(base, HALF)]

        rdma_b = []
        for c in range(NCHUNK):
            rows = pl.ds(base + c * CROWS, CROWS)
            rdma_a[c].wait_recv()
            y = t_ref[pl.ds(c * CROWS, CROWS)] + comm_ref[c]
            ms = jnp.mean(y * y, axis=-1, keepdims=True)
            ob_ref[c] = y * lax.rsqrt(ms + 1e-6) * g_ref[...]
            if PROBE == "full":
                b = pltpu.make_async_remote_copy(
                    src_ref=ob_ref.at[c],
                    dst_ref=o_ref.at[rows],
                    send_sem=send_b.at[c],
                    recv_sem=recv_b.at[c],
                    device_id=x_nbr,
                    device_id_type=pl.DeviceIdType.MESH,
                )
                b.start()
                rdma_b.append(b)

        o_ref[pl.ds(base, HALF)] = ob_ref[...].reshape(HALF, D)
        if PROBE == "aonly":
            o_ref[pl.ds((1 - my_x) * HALF, HALF)] = ob_ref[...].reshape(HALF, D)

        for c in range(NCHUNK):
            rdma_a[c].wait_send()
        for c in range(NCHUNK):
            if PROBE == "full":
                rdma_b[c].wait()

    return pl.pallas_call(
        body,
        out_shape=jax.ShapeDtypeStruct((M, D), jnp.float32),
        in_specs=[
            pl.BlockSpec(memory_space=pltpu.VMEM),
            pl.BlockSpec(memory_space=pltpu.VMEM),
            pl.BlockSpec(memory_space=pltpu.VMEM),
        ],
        out_specs=pl.BlockSpec(memory_space=pltpu.VMEM),
        scratch_shapes=[
            pltpu.VMEM((NCHUNK, CROWS, D), jnp.float32),
            pltpu.VMEM((HALF, D), jnp.float32),
            pltpu.VMEM((NCHUNK, CROWS, D), jnp.float32),
            pltpu.SemaphoreType.DMA((NCHUNK,)),
            pltpu.SemaphoreType.DMA((NCHUNK,)),
            pltpu.SemaphoreType.DMA((NCHUNK,)),
            pltpu.SemaphoreType.DMA((NCHUNK,)),
        ],
        compiler_params=pltpu.CompilerParams(collective_id=0),
    )(partial, resid, gamma)
